# Initial kernel scaffold; baseline (speedup 1.0000x reference)
import jax
import jax.numpy as jnp
from jax.experimental import pallas as pl


def kernel(z, labels):
    raise NotImplementedError("write your pallas kernel here")


# speedup vs baseline: 17.6927x; 17.6927x over previous
import functools

import jax
import jax.numpy as jnp
from jax import lax
from jax.experimental import pallas as pl
from jax.experimental.pallas import tpu as pltpu
from jax.experimental.pallas import tpu_sc as plsc

_K = 64
_D = 128
_L = 16
_NV = _D // _L
_N = 320000
_NC = 2
_NS = 16
_NW = _NC * _NS
_R = _N // _NW
_C = 400
_NCH = _R // _C
_G = _C // _L


def _sc_body(z_hbm, lab_hbm, sums_out, sq_out, cnt_out,
             zb0, zb1, lb0, lb1, acc, sqa, cnta,
             sem_z0, sem_z1, sem_l0, sem_l1):
  zero = jnp.zeros((_L,), jnp.float32)
  wid = lax.axis_index("s") * _NC + lax.axis_index("c")
  base = wid * _R

  def zero_body(k, carry):
    for j in range(_NV):
      acc[k, pl.ds(_L * j, _L)] = zero
    sqa[k, :] = zero
    cnta[k, :] = zero
    return carry
  lax.fori_loop(0, _K, zero_body, 0)

  def start(g, zb, lb, sz, sl):
    off = base + g * _C
    pltpu.make_async_copy(z_hbm.at[pl.ds(off, _C)], zb, sz).start()
    pltpu.make_async_copy(lab_hbm.at[pl.ds(off, _C)], lb, sl).start()

  def wait(zb, lb, sz, sl):
    pltpu.make_async_copy(z_hbm.at[pl.ds(base, _C)], zb, sz).wait()
    pltpu.make_async_copy(lab_hbm.at[pl.ds(base, _C)], lb, sl).wait()

  start(0, zb0, lb0, sem_z0, sem_l0)
  start(1, zb1, lb1, sem_z1, sem_l1)

  one = jnp.full((_L,), 1.0, jnp.float32)
  sixteen = jnp.full((_L,), float(_L), jnp.float32)

  def tree8(parts):
    s0 = (parts[0] + parts[1]) + (parts[2] + parts[3])
    s1 = (parts[4] + parts[5]) + (parts[6] + parts[7])
    return s0 + s1

  def process(zb, lb):
    def group(t, carry):
      lv = lb[pl.ds(_L * t, _L)]
      first = lv[0]
      last = lv[_L - 1]
      row0 = _L * t

      @pl.when(first == last)
      def _():
        accs = [zero] * _NV
        sqs = [zero] * _NV
        for r in range(_L):
          for j in range(_NV):
            v = zb[row0 + r, pl.ds(_L * j, _L)]
            accs[j] = accs[j] + v
            sqs[j] = sqs[j] + v * v
        for j in range(_NV):
          acc[first, pl.ds(_L * j, _L)] = (
              acc[first, pl.ds(_L * j, _L)] + accs[j])
        sqa[first, :] = sqa[first, :] + tree8(sqs)
        cnta[first, :] = cnta[first, :] + sixteen

      @pl.when(first != last)
      def _():
        for r in range(_L):
          labv = lv[r]
          sq_parts = []
          for j in range(_NV):
            v = zb[row0 + r, pl.ds(_L * j, _L)]
            acc[labv, pl.ds(_L * j, _L)] = (
                acc[labv, pl.ds(_L * j, _L)] + v)
            sq_parts.append(v * v)
          sqa[labv, :] = sqa[labv, :] + tree8(sq_parts)
          cnta[labv, :] = cnta[labv, :] + one
      return carry
    lax.fori_loop(0, _G, group, 0)

  def outer(i, carry):
    g0 = 2 * i
    wait(zb0, lb0, sem_z0, sem_l0)
    process(zb0, lb0)

    @pl.when(g0 + 2 < _NCH)
    def _():
      start(g0 + 2, zb0, lb0, sem_z0, sem_l0)

    wait(zb1, lb1, sem_z1, sem_l1)
    process(zb1, lb1)

    @pl.when(g0 + 3 < _NCH)
    def _():
      start(g0 + 3, zb1, lb1, sem_z1, sem_l1)
    return carry

  lax.fori_loop(0, _NCH // 2, outer, 0)
  wait(zb0, lb0, sem_z0, sem_l0)
  process(zb0, lb0)

  pltpu.sync_copy(acc, sums_out.at[wid])
  pltpu.sync_copy(sqa, sq_out.at[wid])
  pltpu.sync_copy(cnta, cnt_out.at[wid])


_sc_stats = pl.kernel(
    _sc_body,
    out_type=[
        jax.ShapeDtypeStruct((_NW, _K, _D), jnp.float32),
        jax.ShapeDtypeStruct((_NW, _K, _L), jnp.float32),
        jax.ShapeDtypeStruct((_NW, _K, _L), jnp.float32),
    ],
    mesh=plsc.VectorSubcoreMesh(core_axis_name="c", subcore_axis_name="s"),
    scratch_types=[
        pltpu.VMEM((_C, _D), jnp.float32),
        pltpu.VMEM((_C, _D), jnp.float32),
        pltpu.VMEM((_C,), jnp.int32),
        pltpu.VMEM((_C,), jnp.int32),
        pltpu.VMEM((_K, _D), jnp.float32),
        pltpu.VMEM((_K, _L), jnp.float32),
        pltpu.VMEM((_K, _L), jnp.float32),
        pltpu.SemaphoreType.DMA,
        pltpu.SemaphoreType.DMA,
        pltpu.SemaphoreType.DMA,
        pltpu.SemaphoreType.DMA,
    ],
)


def _tc_body(sums_ref, sq_ref, cnt_ref, out_ref):
  sums = jnp.sum(sums_ref[...], axis=0)
  sq = jnp.sum(jnp.sum(sq_ref[...], axis=0), axis=1)
  cnt = jnp.sum(jnp.sum(cnt_ref[...], axis=0), axis=1) / _L
  denom = jnp.maximum(cnt, 1.0)
  centers = sums / denom[:, None]
  var = sq / denom - jnp.sum(centers * centers, axis=1)
  valid = cnt > 1.0
  nv = jnp.maximum(jnp.sum(valid.astype(jnp.float32)), 1.0)
  intra = jnp.sum(jnp.where(valid, var, 0.0)) / nv
  dd = centers[:, None, :] - centers[None, :, :]
  dist = jnp.sqrt(jnp.sum(dd * dd, axis=-1))
  inter = -jnp.sum(dist) / (_K * (_K - 1))
  out_ref[...] = jnp.full((1, 1), intra + 0.5 * inter, jnp.float32)


def kernel(z, labels):
  sums_p, sq_p, cnt_p = _sc_stats(z, labels)
  loss = pl.pallas_call(
      _tc_body,
      out_shape=jax.ShapeDtypeStruct((1, 1), jnp.float32),
  )(sums_p, sq_p, cnt_p)
  return loss[0, 0]
